# Initial kernel scaffold; baseline (speedup 1.0000x reference)
import jax
import jax.numpy as jnp
from jax.experimental import pallas as pl


def kernel(x, W_enc, b_enc, W_dec, b_dec):
    raise NotImplementedError("write your pallas kernel here")


# speedup vs baseline: 2.9943x; 2.9943x over previous
import functools

import jax
import jax.numpy as jnp
from jax import lax
from jax.experimental import pallas as pl
from jax.experimental.pallas import tpu as pltpu

D_IN = 768
NUM_LATENTS = 6144
K = 16
B = 2048
BB = 128
NB = B // BB

_NT = (((1,), (1,)), ((), ()))
_NN = (((1,), (0,)), ((), ()))

_SC1 = 2.0 ** 12
_SC2 = 2.0 ** 24
_INV1 = 2.0 ** -12
_INV2 = 2.0 ** -24


def _mp_block(x_ref, wenc_ref, benc_ref, whi_ref, wlo_ref, wlo2_ref, bdec_ref,
              sae_ref, acts_ref, idx_ref, l2_ref, xsum_ref, xsq_ref):
    x = x_ref[...]
    bdec = bdec_ref[...]
    benc = benc_ref[...]
    wenc = wenc_ref[...]
    whi = whi_ref[...]
    wlo = wlo_ref[...]
    wlo2 = wlo2_ref[...]
    resid = x - bdec
    iota = lax.broadcasted_iota(jnp.int32, (BB, NUM_LATENTS), 1)
    l2 = jnp.zeros((), jnp.float32)
    for i in range(K):
        z = lax.dot_general(resid.astype(jnp.bfloat16), wenc, _NT,
                            preferred_element_type=jnp.float32) + benc
        m = jnp.max(z, axis=1, keepdims=True)
        idx = jnp.min(jnp.where(z == m, iota, NUM_LATENTS),
                      axis=1)
        act = jnp.maximum(m, 0.0)
        acts_ref[i, :] = act[:, 0]
        idx_ref[i, :] = idx
        onehot = jnp.where(iota == idx[:, None],
                           1.0, 0.0).astype(jnp.bfloat16)
        wsel = lax.dot_general(onehot, whi, _NN,
                               preferred_element_type=jnp.float32)
        wsel = wsel + lax.dot_general(onehot, wlo, _NN,
                                      preferred_element_type=jnp.float32) * _INV1
        wsel = wsel + lax.dot_general(onehot, wlo2, _NN,
                                      preferred_element_type=jnp.float32) * _INV2
        dec = act * wsel
        sae = dec + bdec
        sae_ref[i, :, :] = sae
        e = sae - x
        l2 = l2 + jnp.sum(e * e)
        if i != K - 1:
            resid = resid - dec
    l2_ref[0, :, :] = l2.reshape(1, 1)
    xsum_ref[0, :, :] = jnp.sum(x, axis=0, keepdims=True)
    xsq_ref[0, :, :] = jnp.sum(x * x).reshape(1, 1)


@jax.jit
def _run(x, W_enc, b_enc, W_dec, b_dec):
    wenc_bf = W_enc.astype(jnp.bfloat16)
    whi = lax.optimization_barrier(W_dec.astype(jnp.bfloat16))
    lo32 = W_dec - whi.astype(jnp.float32)
    wlo = lax.optimization_barrier((lo32 * _SC1).astype(jnp.bfloat16))
    wlo2 = ((lo32 - wlo.astype(jnp.float32) * _INV1) * _SC2).astype(jnp.bfloat16)

    out_shapes = (
        jax.ShapeDtypeStruct((K, B, D_IN), jnp.float32),
        jax.ShapeDtypeStruct((K, B), jnp.float32),
        jax.ShapeDtypeStruct((K, B), jnp.int32),
        jax.ShapeDtypeStruct((NB, 1, 1), jnp.float32),
        jax.ShapeDtypeStruct((NB, 1, D_IN), jnp.float32),
        jax.ShapeDtypeStruct((NB, 1, 1), jnp.float32),
    )
    grid = (NB,)
    sae_out, acts, idx, l2p, xsum, xsq = pl.pallas_call(
        _mp_block,
        grid=grid,
        in_specs=[
            pl.BlockSpec((BB, D_IN), lambda i: (i, 0)),
            pl.BlockSpec((NUM_LATENTS, D_IN), lambda i: (0, 0)),
            pl.BlockSpec((1, NUM_LATENTS), lambda i: (0, 0)),
            pl.BlockSpec((NUM_LATENTS, D_IN), lambda i: (0, 0)),
            pl.BlockSpec((NUM_LATENTS, D_IN), lambda i: (0, 0)),
            pl.BlockSpec((NUM_LATENTS, D_IN), lambda i: (0, 0)),
            pl.BlockSpec((1, D_IN), lambda i: (0, 0)),
        ],
        out_specs=(
            pl.BlockSpec((K, BB, D_IN), lambda i: (0, i, 0)),
            pl.BlockSpec((K, BB), lambda i: (0, i)),
            pl.BlockSpec((K, BB), lambda i: (0, i)),
            pl.BlockSpec((1, 1, 1), lambda i: (i, 0, 0)),
            pl.BlockSpec((1, 1, D_IN), lambda i: (i, 0, 0)),
            pl.BlockSpec((1, 1, 1), lambda i: (i, 0, 0)),
        ),
        out_shape=out_shapes,
    )(x, wenc_bf, b_enc.reshape(1, NUM_LATENTS), whi, wlo, wlo2,
      b_dec.reshape(1, D_IN))

    l2_loss = jnp.sum(l2p)
    colsum = jnp.sum(xsum, axis=(0, 1))
    total_variance = jnp.sum(xsq) - jnp.sum(colsum * colsum) / B
    fvu = l2_loss / total_variance
    return (sae_out, acts.reshape(K, B, 1), idx.reshape(K, B, 1), fvu)


def kernel(x, W_enc, b_enc, W_dec, b_dec):
    return _run(x, W_enc, b_enc, W_dec, b_dec)


# speedup vs baseline: 3.1065x; 1.0375x over previous
import functools

import jax
import jax.numpy as jnp
from jax import lax
from jax.experimental import pallas as pl
from jax.experimental.pallas import tpu as pltpu

D_IN = 768
NUM_LATENTS = 6144
K = 16
B = 2048
BB = 128
NB = B // BB

_NT = (((1,), (1,)), ((), ()))
_NN = (((1,), (0,)), ((), ()))

_SC1 = 2.0 ** 12
_SC2 = 2.0 ** 24
_INV1 = 2.0 ** -12
_INV2 = 2.0 ** -24


def _mp_block(x_ref, wenc_ref, benc_ref, whi_ref, wlo_ref, wlo2_ref, bdec_ref,
              sae_ref, acts_ref, idx_ref, l2_ref, xsum_ref, xsq_ref):
    x = x_ref[...]
    bdec = bdec_ref[...]
    benc = benc_ref[...]
    wenc = wenc_ref[...]
    whi = whi_ref[...]
    wlo = wlo_ref[...]
    wlo2 = wlo2_ref[...]
    resid = x - bdec
    iota = lax.broadcasted_iota(jnp.int32, (BB, NUM_LATENTS), 1)
    l2 = jnp.zeros((), jnp.float32)
    for i in range(K):
        z = lax.dot_general(resid.astype(jnp.bfloat16), wenc, _NT,
                            preferred_element_type=jnp.float32) + benc
        m = jnp.max(z, axis=1, keepdims=True)
        idx = jnp.argmax(z, axis=1).astype(jnp.int32)
        act = jnp.maximum(m, 0.0)
        acts_ref[i, :] = act[:, 0]
        idx_ref[i, :] = idx
        onehot = jnp.where(iota == idx[:, None],
                           1.0, 0.0).astype(jnp.bfloat16)
        wsel = lax.dot_general(onehot, whi, _NN,
                               preferred_element_type=jnp.float32)
        wsel = wsel + lax.dot_general(onehot, wlo, _NN,
                                      preferred_element_type=jnp.float32) * _INV1
        wsel = wsel + lax.dot_general(onehot, wlo2, _NN,
                                      preferred_element_type=jnp.float32) * _INV2
        dec = act * wsel
        sae = dec + bdec
        sae_ref[i, :, :] = sae
        e = sae - x
        l2 = l2 + jnp.sum(e * e)
        if i != K - 1:
            resid = resid - dec
    l2_ref[0, :, :] = l2.reshape(1, 1)
    xsum_ref[0, :, :] = jnp.sum(x, axis=0, keepdims=True)
    xsq_ref[0, :, :] = jnp.sum(x * x).reshape(1, 1)


@jax.jit
def _run(x, W_enc, b_enc, W_dec, b_dec):
    wenc_bf = W_enc.astype(jnp.bfloat16)
    whi = lax.optimization_barrier(W_dec.astype(jnp.bfloat16))
    lo32 = W_dec - whi.astype(jnp.float32)
    wlo = lax.optimization_barrier((lo32 * _SC1).astype(jnp.bfloat16))
    wlo2 = ((lo32 - wlo.astype(jnp.float32) * _INV1) * _SC2).astype(jnp.bfloat16)

    out_shapes = (
        jax.ShapeDtypeStruct((K, B, D_IN), jnp.float32),
        jax.ShapeDtypeStruct((K, B), jnp.float32),
        jax.ShapeDtypeStruct((K, B), jnp.int32),
        jax.ShapeDtypeStruct((NB, 1, 1), jnp.float32),
        jax.ShapeDtypeStruct((NB, 1, D_IN), jnp.float32),
        jax.ShapeDtypeStruct((NB, 1, 1), jnp.float32),
    )
    grid = (NB,)
    sae_out, acts, idx, l2p, xsum, xsq = pl.pallas_call(
        _mp_block,
        grid=grid,
        in_specs=[
            pl.BlockSpec((BB, D_IN), lambda i: (i, 0)),
            pl.BlockSpec((NUM_LATENTS, D_IN), lambda i: (0, 0)),
            pl.BlockSpec((1, NUM_LATENTS), lambda i: (0, 0)),
            pl.BlockSpec((NUM_LATENTS, D_IN), lambda i: (0, 0)),
            pl.BlockSpec((NUM_LATENTS, D_IN), lambda i: (0, 0)),
            pl.BlockSpec((NUM_LATENTS, D_IN), lambda i: (0, 0)),
            pl.BlockSpec((1, D_IN), lambda i: (0, 0)),
        ],
        out_specs=(
            pl.BlockSpec((K, BB, D_IN), lambda i: (0, i, 0)),
            pl.BlockSpec((K, BB), lambda i: (0, i)),
            pl.BlockSpec((K, BB), lambda i: (0, i)),
            pl.BlockSpec((1, 1, 1), lambda i: (i, 0, 0)),
            pl.BlockSpec((1, 1, D_IN), lambda i: (i, 0, 0)),
            pl.BlockSpec((1, 1, 1), lambda i: (i, 0, 0)),
        ),
        out_shape=out_shapes,
    )(x, wenc_bf, b_enc.reshape(1, NUM_LATENTS), whi, wlo, wlo2,
      b_dec.reshape(1, D_IN))

    l2_loss = jnp.sum(l2p)
    colsum = jnp.sum(xsum, axis=(0, 1))
    total_variance = jnp.sum(xsq) - jnp.sum(colsum * colsum) / B
    fvu = l2_loss / total_variance
    return (sae_out, acts.reshape(K, B, 1), idx.reshape(K, B, 1), fvu)


def kernel(x, W_enc, b_enc, W_dec, b_dec):
    return _run(x, W_enc, b_enc, W_dec, b_dec)


# speedup vs baseline: 3.6935x; 1.1890x over previous
import functools

import jax
import jax.numpy as jnp
from jax import lax
from jax.experimental import pallas as pl
from jax.experimental.pallas import tpu as pltpu
from jax.experimental.pallas import tpu_sc as plsc

D_IN = 768
NUM_LATENTS = 6144
K = 16
B = 2048
BB = 256
NB = B // BB
BBC = 256
NBC = B // BBC

_NT = (((1,), (1,)), ((), ()))
_NN = (((1,), (0,)), ((), ()))

_SC1 = 2.0 ** 12
_SC2 = 2.0 ** 24
_INV1 = 2.0 ** -12
_INV2 = 2.0 ** -24

_ROWS = K * B
_NW = 32
_RPW = _ROWS // _NW
_CH = 128
_NCH = _RPW // _CH


def _mp_block(x_ref, wenc_ref, benc_ref, whi_ref, wlo_ref, wlo2_ref, bdec_ref,
              acts_ref, idx_ref):
    x = x_ref[...]
    bdec = bdec_ref[...]
    benc = benc_ref[...]
    wenc = wenc_ref[...]
    whi = whi_ref[...]
    wlo = wlo_ref[...]
    wlo2 = wlo2_ref[...]
    resid = x - bdec
    iota = lax.broadcasted_iota(jnp.int32, (BB, NUM_LATENTS), 1)
    for i in range(K):
        z = lax.dot_general(resid.astype(jnp.bfloat16), wenc, _NT,
                            preferred_element_type=jnp.float32) + benc
        m = jnp.max(z, axis=1, keepdims=True)
        idx = jnp.argmax(z, axis=1).astype(jnp.int32)
        act = jnp.maximum(m, 0.0)
        acts_ref[i, :] = act[:, 0]
        idx_ref[i, :] = idx
        if i != K - 1:
            onehot = jnp.where(iota == idx[:, None],
                               1.0, 0.0).astype(jnp.bfloat16)
            wsel = lax.dot_general(onehot, whi, _NN,
                                   preferred_element_type=jnp.float32)
            wsel = wsel + lax.dot_general(onehot, wlo, _NN,
                                          preferred_element_type=jnp.float32) * _INV1
            wsel = wsel + lax.dot_general(onehot, wlo2, _NN,
                                          preferred_element_type=jnp.float32) * _INV2
            resid = resid - act * wsel


def _sc_gather(table_hbm, idx_hbm, out_hbm, idx_v, rows_v, sem):
    wid = lax.axis_index("s") * 2 + lax.axis_index("c")
    base = wid * _RPW
    for c in range(_NCH):
        off = base + c * _CH
        pltpu.sync_copy(idx_hbm.at[pl.ds(off, _CH)], idx_v)
        pltpu.async_copy(table_hbm.at[idx_v], rows_v, sem).wait()
        pltpu.sync_copy(rows_v, out_hbm.at[pl.ds(off, _CH)])


def _epilogue(rows_ref, act_ref, x_ref, bdec_ref,
              sae_ref, l2_ref, xsum_ref, xsq_ref):
    k = pl.program_id(1)
    x = x_ref[...]
    rows = rows_ref[0]
    act = act_ref[0, 0]
    sae = act[:, None] * rows + bdec_ref[...]
    sae_ref[0] = sae
    e = sae - x
    l2_ref[0, 0] = jnp.sum(e * e).reshape(1, 1)

    @pl.when(k == 0)
    def _():
        xsum_ref[0, :, :] = jnp.sum(x, axis=0, keepdims=True)
        xsq_ref[0, :, :] = jnp.sum(x * x).reshape(1, 1)


@jax.jit
def _run(x, W_enc, b_enc, W_dec, b_dec):
    wenc_bf = W_enc.astype(jnp.bfloat16)
    whi = lax.optimization_barrier(W_dec.astype(jnp.bfloat16))
    lo32 = W_dec - whi.astype(jnp.float32)
    wlo = lax.optimization_barrier((lo32 * _SC1).astype(jnp.bfloat16))
    wlo2 = ((lo32 - wlo.astype(jnp.float32) * _INV1) * _SC2).astype(jnp.bfloat16)

    acts, idx = pl.pallas_call(
        _mp_block,
        grid=(NB,),
        in_specs=[
            pl.BlockSpec((BB, D_IN), lambda i: (i, 0)),
            pl.BlockSpec((NUM_LATENTS, D_IN), lambda i: (0, 0)),
            pl.BlockSpec((1, NUM_LATENTS), lambda i: (0, 0)),
            pl.BlockSpec((NUM_LATENTS, D_IN), lambda i: (0, 0)),
            pl.BlockSpec((NUM_LATENTS, D_IN), lambda i: (0, 0)),
            pl.BlockSpec((NUM_LATENTS, D_IN), lambda i: (0, 0)),
            pl.BlockSpec((1, D_IN), lambda i: (0, 0)),
        ],
        out_specs=(
            pl.BlockSpec((K, BB), lambda i: (0, i)),
            pl.BlockSpec((K, BB), lambda i: (0, i)),
        ),
        out_shape=(
            jax.ShapeDtypeStruct((K, B), jnp.float32),
            jax.ShapeDtypeStruct((K, B), jnp.int32),
        ),
    )(x, wenc_bf, b_enc.reshape(1, NUM_LATENTS), whi, wlo, wlo2,
      b_dec.reshape(1, D_IN))

    idx_flat = idx.reshape(_ROWS)
    mesh = plsc.VectorSubcoreMesh(core_axis_name="c", subcore_axis_name="s")
    rows = pl.kernel(
        _sc_gather,
        out_type=jax.ShapeDtypeStruct((_ROWS, D_IN), jnp.float32),
        mesh=mesh,
        scratch_types=[
            pltpu.VMEM((_CH,), jnp.int32),
            pltpu.VMEM((_CH, D_IN), jnp.float32),
            pltpu.SemaphoreType.DMA,
        ],
    )(W_dec, idx_flat)

    rows_kbd = rows.reshape(K, B, D_IN)
    sae_out, l2p, xsum, xsq = pl.pallas_call(
        _epilogue,
        grid=(NBC, K),
        in_specs=[
            pl.BlockSpec((1, BBC, D_IN), lambda j, k: (k, j, 0)),
            pl.BlockSpec((1, 1, BBC), lambda j, k: (k, 0, j)),
            pl.BlockSpec((BBC, D_IN), lambda j, k: (j, 0)),
            pl.BlockSpec((1, D_IN), lambda j, k: (0, 0)),
        ],
        out_specs=(
            pl.BlockSpec((1, BBC, D_IN), lambda j, k: (k, j, 0)),
            pl.BlockSpec((1, 1, 1, 1), lambda j, k: (j, k, 0, 0)),
            pl.BlockSpec((1, 1, D_IN), lambda j, k: (j, 0, 0)),
            pl.BlockSpec((1, 1, 1), lambda j, k: (j, 0, 0)),
        ),
        out_shape=(
            jax.ShapeDtypeStruct((K, B, D_IN), jnp.float32),
            jax.ShapeDtypeStruct((NBC, K, 1, 1), jnp.float32),
            jax.ShapeDtypeStruct((NBC, 1, D_IN), jnp.float32),
            jax.ShapeDtypeStruct((NBC, 1, 1), jnp.float32),
        ),
    )(rows_kbd, acts.reshape(K, 1, B), x, b_dec.reshape(1, D_IN))

    l2_loss = jnp.sum(l2p)
    colsum = jnp.sum(xsum, axis=(0, 1))
    total_variance = jnp.sum(xsq) - jnp.sum(colsum * colsum) / B
    fvu = l2_loss / total_variance
    return (sae_out, acts.reshape(K, B, 1), idx.reshape(K, B, 1), fvu)


def kernel(x, W_enc, b_enc, W_dec, b_dec):
    return _run(x, W_enc, b_enc, W_dec, b_dec)


# speedup vs baseline: 3.8534x; 1.0433x over previous
import functools

import jax
import jax.numpy as jnp
from jax import lax
from jax.experimental import pallas as pl
from jax.experimental.pallas import tpu as pltpu
from jax.experimental.pallas import tpu_sc as plsc

D_IN = 768
NUM_LATENTS = 6144
K = 16
B = 2048
BB = 512
NB = B // BB
BBC = 256
NBC = B // BBC

_NT = (((1,), (1,)), ((), ()))
_NN = (((1,), (0,)), ((), ()))

_SC1 = 2.0 ** 12
_SC2 = 2.0 ** 24
_INV1 = 2.0 ** -12
_INV2 = 2.0 ** -24

_ROWS = K * B
_NW = 32
_RPW = _ROWS // _NW
_CH = 128
_NCH = _RPW // _CH


def _mp_block(x_ref, wenc_ref, benc_ref, whi_ref, wlo_ref, wlo2_ref, bdec_ref,
              acts_ref, idx_ref):
    x = x_ref[...]
    bdec = bdec_ref[...]
    benc = benc_ref[...]
    wenc = wenc_ref[...]
    whi = whi_ref[...]
    wlo = wlo_ref[...]
    wlo2 = wlo2_ref[...]
    resid = x - bdec
    iota = lax.broadcasted_iota(jnp.int32, (BB, NUM_LATENTS), 1)
    for i in range(K):
        z = lax.dot_general(resid.astype(jnp.bfloat16), wenc, _NT,
                            preferred_element_type=jnp.float32) + benc
        m = jnp.max(z, axis=1, keepdims=True)
        idx = jnp.argmax(z, axis=1).astype(jnp.int32)
        act = jnp.maximum(m, 0.0)
        acts_ref[i, :] = act[:, 0]
        idx_ref[i, :] = idx
        if i != K - 1:
            onehot = jnp.where(iota == idx[:, None],
                               1.0, 0.0).astype(jnp.bfloat16)
            wsel = lax.dot_general(onehot, whi, _NN,
                                   preferred_element_type=jnp.float32)
            wsel = wsel + lax.dot_general(onehot, wlo, _NN,
                                          preferred_element_type=jnp.float32) * _INV1
            wsel = wsel + lax.dot_general(onehot, wlo2, _NN,
                                          preferred_element_type=jnp.float32) * _INV2
            resid = resid - act * wsel


def _sc_gather(table_hbm, idx_hbm, out_hbm, idx_v, rows_v, sem):
    wid = lax.axis_index("s") * 2 + lax.axis_index("c")
    base = wid * _RPW
    for c in range(_NCH):
        off = base + c * _CH
        pltpu.sync_copy(idx_hbm.at[pl.ds(off, _CH)], idx_v)
        pltpu.async_copy(table_hbm.at[idx_v], rows_v, sem).wait()
        pltpu.sync_copy(rows_v, out_hbm.at[pl.ds(off, _CH)])


def _epilogue(rows_ref, act_ref, x_ref, bdec_ref,
              sae_ref, l2_ref, xsum_ref, xsq_ref):
    k = pl.program_id(1)
    x = x_ref[...]
    rows = rows_ref[0]
    act = act_ref[0, 0]
    sae = act[:, None] * rows + bdec_ref[...]
    sae_ref[0] = sae
    e = sae - x
    l2_ref[0, 0] = jnp.sum(e * e).reshape(1, 1)

    @pl.when(k == 0)
    def _():
        xsum_ref[0, :, :] = jnp.sum(x, axis=0, keepdims=True)
        xsq_ref[0, :, :] = jnp.sum(x * x).reshape(1, 1)


@jax.jit
def _run(x, W_enc, b_enc, W_dec, b_dec):
    wenc_bf = W_enc.astype(jnp.bfloat16)
    whi = lax.optimization_barrier(W_dec.astype(jnp.bfloat16))
    lo32 = W_dec - whi.astype(jnp.float32)
    wlo = lax.optimization_barrier((lo32 * _SC1).astype(jnp.bfloat16))
    wlo2 = ((lo32 - wlo.astype(jnp.float32) * _INV1) * _SC2).astype(jnp.bfloat16)

    acts, idx = pl.pallas_call(
        _mp_block,
        grid=(NB,),
        in_specs=[
            pl.BlockSpec((BB, D_IN), lambda i: (i, 0)),
            pl.BlockSpec((NUM_LATENTS, D_IN), lambda i: (0, 0)),
            pl.BlockSpec((1, NUM_LATENTS), lambda i: (0, 0)),
            pl.BlockSpec((NUM_LATENTS, D_IN), lambda i: (0, 0)),
            pl.BlockSpec((NUM_LATENTS, D_IN), lambda i: (0, 0)),
            pl.BlockSpec((NUM_LATENTS, D_IN), lambda i: (0, 0)),
            pl.BlockSpec((1, D_IN), lambda i: (0, 0)),
        ],
        out_specs=(
            pl.BlockSpec((K, BB), lambda i: (0, i)),
            pl.BlockSpec((K, BB), lambda i: (0, i)),
        ),
        out_shape=(
            jax.ShapeDtypeStruct((K, B), jnp.float32),
            jax.ShapeDtypeStruct((K, B), jnp.int32),
        ),
    )(x, wenc_bf, b_enc.reshape(1, NUM_LATENTS), whi, wlo, wlo2,
      b_dec.reshape(1, D_IN))

    idx_flat = idx.reshape(_ROWS)
    mesh = plsc.VectorSubcoreMesh(core_axis_name="c", subcore_axis_name="s")
    rows = pl.kernel(
        _sc_gather,
        out_type=jax.ShapeDtypeStruct((_ROWS, D_IN), jnp.float32),
        mesh=mesh,
        scratch_types=[
            pltpu.VMEM((_CH,), jnp.int32),
            pltpu.VMEM((_CH, D_IN), jnp.float32),
            pltpu.SemaphoreType.DMA,
        ],
    )(W_dec, idx_flat)

    rows_kbd = rows.reshape(K, B, D_IN)
    sae_out, l2p, xsum, xsq = pl.pallas_call(
        _epilogue,
        grid=(NBC, K),
        in_specs=[
            pl.BlockSpec((1, BBC, D_IN), lambda j, k: (k, j, 0)),
            pl.BlockSpec((1, 1, BBC), lambda j, k: (k, 0, j)),
            pl.BlockSpec((BBC, D_IN), lambda j, k: (j, 0)),
            pl.BlockSpec((1, D_IN), lambda j, k: (0, 0)),
        ],
        out_specs=(
            pl.BlockSpec((1, BBC, D_IN), lambda j, k: (k, j, 0)),
            pl.BlockSpec((1, 1, 1, 1), lambda j, k: (j, k, 0, 0)),
            pl.BlockSpec((1, 1, D_IN), lambda j, k: (j, 0, 0)),
            pl.BlockSpec((1, 1, 1), lambda j, k: (j, 0, 0)),
        ),
        out_shape=(
            jax.ShapeDtypeStruct((K, B, D_IN), jnp.float32),
            jax.ShapeDtypeStruct((NBC, K, 1, 1), jnp.float32),
            jax.ShapeDtypeStruct((NBC, 1, D_IN), jnp.float32),
            jax.ShapeDtypeStruct((NBC, 1, 1), jnp.float32),
        ),
    )(rows_kbd, acts.reshape(K, 1, B), x, b_dec.reshape(1, D_IN))

    l2_loss = jnp.sum(l2p)
    colsum = jnp.sum(xsum, axis=(0, 1))
    total_variance = jnp.sum(xsq) - jnp.sum(colsum * colsum) / B
    fvu = l2_loss / total_variance
    return (sae_out, acts.reshape(K, B, 1), idx.reshape(K, B, 1), fvu)


def kernel(x, W_enc, b_enc, W_dec, b_dec):
    return _run(x, W_enc, b_enc, W_dec, b_dec)
